# baseline (speedup 1.0000x reference)
import functools

import jax
import jax.numpy as jnp
from jax import lax
from jax.experimental import pallas as pl
from jax.experimental.pallas import tpu as pltpu
from jax.experimental.pallas import tpu_sc as plsc

_NC = 2
_NS = 16
_NW = _NC * _NS
_LANE = 128
_K = 4


@functools.lru_cache(maxsize=None)
def _make_gather(n_rows, rows_w, d):
    mesh = plsc.VectorSubcoreMesh(core_axis_name="c", subcore_axis_name="s")

    @functools.partial(
        pl.kernel,
        mesh=mesh,
        compiler_params=pltpu.CompilerParams(use_tc_tiling_on_sc=False),
        out_type=jax.ShapeDtypeStruct((n_rows, d), jnp.float32),
        scratch_types=[
            pltpu.VMEM((rows_w, _LANE), jnp.int32),
        ]
        + [pltpu.VMEM((_LANE, d), jnp.float32) for _ in range(_K)]
        + [pltpu.SemaphoreType.DMA for _ in range(_K)],
    )
    def gather(w_hbm, idx_hbm, out_hbm, idx_v, *rest):
        ring = rest[:_K]
        sem_g = rest[_K:]

        wid = lax.axis_index("s") * _NC + lax.axis_index("c")
        r0 = wid * rows_w
        pltpu.sync_copy(idx_hbm.at[pl.ds(r0, rows_w)], idx_v)

        def issue_gather(chunk, slot):
            pltpu.async_copy(
                w_hbm.at[idx_v.at[chunk]], ring[slot], sem_g[slot]
            )

        for s in range(_K):
            issue_gather(s, s)

        def step(t, carry):
            for s in range(_K):
                chunk = t * _K + s
                pltpu.make_async_copy(
                    w_hbm.at[pl.ds(0, _LANE)], ring[s], sem_g[s]
                ).wait()
                pltpu.sync_copy(
                    ring[s],
                    out_hbm.at[pl.ds((r0 + chunk) * _LANE, _LANE)],
                )

                @pl.when(chunk + _K < rows_w)
                def _():
                    issue_gather(chunk + _K, s)

            return carry

        lax.fori_loop(0, rows_w // _K, step, 0)

    return gather


def kernel(token_ids, weight):
    b, f = token_ids.shape
    v, d = weight.shape
    n = b * f

    w128 = lax.optimization_barrier(weight.reshape(v * d // _LANE, _LANE))
    wflat = w128.reshape(v, d)

    idx = token_ids.reshape(n).astype(jnp.int32)
    chunk = _NW * _LANE
    n_pad = -(-n // chunk) * chunk
    if n_pad != n:
        idx = jnp.concatenate(
            [idx, jnp.zeros((n_pad - n,), jnp.int32)]
        )

    idx2 = idx.reshape(n_pad // _LANE, _LANE)
    rows_w = (n_pad // _LANE) // _NW

    out = _make_gather(n_pad, rows_w, d)(wflat, idx2)
    if n_pad != n:
        out = out[:n]
    return out.reshape(b, f, d)
